# Initial kernel scaffold; baseline (speedup 1.0000x reference)
import jax
import jax.numpy as jnp
from jax.experimental import pallas as pl


def kernel(x, edge_index, W, a):
    raise NotImplementedError("write your pallas kernel here")


# speedup vs baseline: 8.4190x; 8.4190x over previous
import functools

import jax
import jax.numpy as jnp
from jax import lax
from jax.experimental import pallas as pl
from jax.experimental.pallas import tpu as pltpu
from jax.experimental.pallas import tpu_sc as plsc

IN_F = 128
OUT_F = 128
ALPHA = 0.2
N_NODES = 10000
N_EDGES = 320000

NC = 2
NS = 16
NW = NC * NS
EPW = N_EDGES // NW
CHUNK = 128
NFULL = EPW // CHUNK
TAIL = EPW - NFULL * CHUNK
GROUPS = CHUNK // 16
RSTRIPE = 624
RTAIL = N_NODES - NS * RSTRIPE

BLK = 1000


def _mm_body(x_ref, w_ref, at_ref, wh_ref, s_ref):
    wh = jnp.dot(x_ref[...], w_ref[...], preferred_element_type=jnp.float32)
    wh_ref[...] = wh
    s_ref[...] = lax.dot_general(
        at_ref[...], wh, (((1,), (1,)), ((), ())),
        preferred_element_type=jnp.float32)


def _matmul_scores(x, W, At):
    return pl.pallas_call(
        _mm_body,
        out_shape=[
            jax.ShapeDtypeStruct((N_NODES, OUT_F), jnp.float32),
            jax.ShapeDtypeStruct((IN_F, N_NODES), jnp.float32),
        ],
    )(x, W, At)


def _make_sc_kernel():
    mesh = plsc.VectorSubcoreMesh(core_axis_name="c", subcore_axis_name="s",
                                  num_cores=NC, num_subcores=NS)

    @functools.partial(
        pl.kernel,
        out_type=[
            jax.ShapeDtypeStruct((NC, N_NODES, OUT_F), jnp.float32),
            jax.ShapeDtypeStruct((NW, 1, 16), jnp.float32),
        ],
        mesh=mesh,
        scratch_types=[
            pltpu.VMEM((N_NODES,), jnp.float32),
            pltpu.VMEM((N_NODES,), jnp.float32),
            pltpu.VMEM((CHUNK,), jnp.int32),
            pltpu.VMEM((CHUNK,), jnp.int32),
            pltpu.VMEM((CHUNK, OUT_F), jnp.float32),
            pltpu.VMEM((CHUNK,), jnp.float32),
            pltpu.VMEM((16,), jnp.float32),
            pltpu.VMEM_SHARED((N_NODES, OUT_F), jnp.float32),
            pltpu.SemaphoreType.DMA,
        ],
        compiler_params=pltpu.CompilerParams(needs_layout_passes=False),
    )
    def sc_kernel(wh_hbm, s_hbm, esrc_hbm, edst_hbm, parts_hbm, psums_hbm,
                  s1_v, s2_v, sidx_v, didx_v, rows_v, p_v, psum_v, acc, sem):
        cid = lax.axis_index("c")
        sid = lax.axis_index("s")
        wid = sid * NC + cid

        pltpu.sync_copy(s_hbm.at[0], s1_v)
        pltpu.sync_copy(s_hbm.at[1], s2_v)

        def _vmax(ref):
            def body(i, m):
                return jnp.maximum(m, ref[pl.ds(i * 16, 16)])
            m = lax.fori_loop(0, N_NODES // 16, body,
                              jnp.full((16,), -jnp.inf, jnp.float32))
            r = m[0]
            for i in range(1, 16):
                r = jnp.maximum(r, m[i])
            return r

        mb = _vmax(s1_v) + _vmax(s2_v)
        c_shift = jnp.where(mb >= 0, mb, ALPHA * mb)

        zero16 = jnp.zeros((16,), jnp.float32)

        def zrow(i, carry):
            for j in range(GROUPS):
                rows_v[i, pl.ds(j * 16, 16)] = zero16
            return carry

        lax.fori_loop(0, CHUNK, zrow, 0)
        zbase = sid * RSTRIPE
        for kk in range(4):
            pltpu.sync_copy(rows_v,
                            acc.at[pl.ds(zbase + kk * CHUNK, CHUNK)])
        pltpu.sync_copy(rows_v.at[pl.ds(0, RSTRIPE - 4 * CHUNK)],
                        acc.at[pl.ds(zbase + 4 * CHUNK,
                                     RSTRIPE - 4 * CHUNK)])

        @pl.when(sid == NS - 1)
        def _zero_tail():
            pltpu.sync_copy(rows_v.at[pl.ds(0, RTAIL)],
                            acc.at[pl.ds(NS * RSTRIPE, RTAIL)])

        plsc.subcore_barrier()

        def weights_for_group(g, psum):
            si = sidx_v[pl.ds(g * 16, 16)]
            di = didx_v[pl.ds(g * 16, 16)]
            e = plsc.load_gather(s1_v, [si]) + plsc.load_gather(s2_v, [di])
            e = jnp.where(e >= 0, e, ALPHA * e)
            p = jnp.exp(e - c_shift)
            p_v[pl.ds(g * 16, 16)] = p
            return psum + p

        def scale_group(g, carry):
            p16 = p_v[pl.ds(g * 16, 16)]
            for j in range(16):
                pe = p16[j]
                ei = g * 16 + j
                for k in range(GROUPS):
                    sl = pl.ds(k * 16, 16)
                    rows_v[ei, sl] = rows_v[ei, sl] * pe
            return carry

        def chunk_body(ci, psum):
            base = wid * EPW + ci * CHUNK
            pltpu.sync_copy(esrc_hbm.at[pl.ds(base, CHUNK)], sidx_v)
            pltpu.sync_copy(edst_hbm.at[pl.ds(base, CHUNK)], didx_v)
            pltpu.async_copy(wh_hbm.at[didx_v], rows_v, sem).wait()
            for g in range(GROUPS):
                psum = weights_for_group(g, psum)
            lax.fori_loop(0, GROUPS, scale_group, 0)
            pltpu.sync_copy(rows_v, acc.at[sidx_v], add=True)
            return psum

        psum = lax.fori_loop(0, NFULL, chunk_body,
                             jnp.zeros((16,), jnp.float32))

        tbase = wid * EPW + NFULL * CHUNK
        pltpu.sync_copy(esrc_hbm.at[pl.ds(tbase, TAIL)],
                        sidx_v.at[pl.ds(0, TAIL)])
        pltpu.sync_copy(edst_hbm.at[pl.ds(tbase, TAIL)],
                        didx_v.at[pl.ds(0, TAIL)])
        pltpu.async_copy(wh_hbm.at[didx_v], rows_v, sem).wait()
        psum = weights_for_group(0, psum)
        for g in range(1, GROUPS):
            p_v[pl.ds(g * 16, 16)] = zero16
        lax.fori_loop(0, GROUPS, scale_group, 0)
        pltpu.sync_copy(rows_v, acc.at[sidx_v], add=True)

        plsc.subcore_barrier()

        pltpu.sync_copy(acc.at[pl.ds(sid * RSTRIPE, RSTRIPE)],
                        parts_hbm.at[cid, pl.ds(sid * RSTRIPE, RSTRIPE)])

        @pl.when(sid == NS - 1)
        def _out_tail():
            pltpu.sync_copy(acc.at[pl.ds(NS * RSTRIPE, RTAIL)],
                            parts_hbm.at[cid, pl.ds(NS * RSTRIPE, RTAIL)])

        psum_v[...] = psum
        pltpu.sync_copy(psum_v, psums_hbm.at[wid, 0])

    return sc_kernel


_sc_kernel_cache = None


def _get_sc_kernel():
    global _sc_kernel_cache
    if _sc_kernel_cache is None:
        _sc_kernel_cache = _make_sc_kernel()
    return _sc_kernel_cache


def _ep_body(parts_ref, psums_ref, o_ref):
    s = jnp.sum(psums_ref[...])
    v = (parts_ref[0] + parts_ref[1]) * (1.0 / s)
    o_ref[...] = jnp.where(v > 0, v, jnp.exp(v) - 1.0)


def _epilogue(parts, psums):
    return pl.pallas_call(
        _ep_body,
        grid=(N_NODES // BLK,),
        in_specs=[
            pl.BlockSpec((NC, BLK, OUT_F), lambda i: (0, i, 0)),
            pl.BlockSpec((NW, 1, 16), lambda i: (0, 0, 0)),
        ],
        out_specs=pl.BlockSpec((BLK, OUT_F), lambda i: (i, 0)),
        out_shape=jax.ShapeDtypeStruct((N_NODES, OUT_F), jnp.float32),
    )(parts, psums)


def kernel(x, edge_index, W, a):
    x = x.astype(jnp.float32)
    ei = edge_index.astype(jnp.int32)
    a_col = a[:, 0].astype(jnp.float32)
    At = jnp.zeros((IN_F, IN_F), jnp.float32)
    At = At.at[0, :].set(a_col[:IN_F]).at[1, :].set(a_col[IN_F:])
    Wh, s_all = _matmul_scores(x, W.astype(jnp.float32), At)
    parts, psums = _get_sc_kernel()(Wh, s_all, ei[0], ei[1])
    return _epilogue(parts, psums)


# speedup vs baseline: 11.4205x; 1.3565x over previous
import functools

import jax
import jax.numpy as jnp
from jax import lax
from jax.experimental import pallas as pl
from jax.experimental.pallas import tpu as pltpu
from jax.experimental.pallas import tpu_sc as plsc

IN_F = 128
OUT_F = 128
ALPHA = 0.2
N_NODES = 10000
N_EDGES = 320000

NC = 2
NS = 16
NW = NC * NS
EPW = N_EDGES // NW
CHUNK = 128
NFULL = EPW // CHUNK
TAIL = EPW - NFULL * CHUNK
GROUPS = CHUNK // 16
SUPER = 2000
NSUP = EPW // SUPER
RSTRIPE = 624
RTAIL = N_NODES - NS * RSTRIPE

BLK = 1000


def _mm_body(x_ref, w_ref, at_ref, wh_ref, s_ref):
    wh = jnp.dot(x_ref[...], w_ref[...], preferred_element_type=jnp.float32)
    wh_ref[...] = wh
    s_ref[...] = lax.dot_general(
        at_ref[...], wh, (((1,), (1,)), ((), ())),
        preferred_element_type=jnp.float32)


def _matmul_scores(x, W, At):
    return pl.pallas_call(
        _mm_body,
        out_shape=[
            jax.ShapeDtypeStruct((N_NODES, OUT_F), jnp.float32),
            jax.ShapeDtypeStruct((IN_F, N_NODES), jnp.float32),
        ],
    )(x, W, At)


def _make_weights_kernel(mesh):
    @functools.partial(
        pl.kernel,
        out_type=[
            jax.ShapeDtypeStruct((N_EDGES,), jnp.float32),
            jax.ShapeDtypeStruct((NW, 1, 16), jnp.float32),
        ],
        mesh=mesh,
        scratch_types=[
            pltpu.VMEM((N_NODES,), jnp.float32),
            pltpu.VMEM((N_NODES,), jnp.float32),
            pltpu.VMEM((SUPER,), jnp.int32),
            pltpu.VMEM((SUPER,), jnp.int32),
            pltpu.VMEM((SUPER,), jnp.float32),
            pltpu.VMEM((16,), jnp.float32),
        ],
        compiler_params=pltpu.CompilerParams(needs_layout_passes=False),
    )
    def weights_kernel(s_hbm, esrc_hbm, edst_hbm, p_hbm, psums_hbm,
                       s1_v, s2_v, sidx_v, didx_v, pb_v, psum_v):
        cid = lax.axis_index("c")
        sid = lax.axis_index("s")
        wid = sid * NC + cid
        ebase = wid * EPW

        pltpu.sync_copy(s_hbm.at[0], s1_v)
        pltpu.sync_copy(s_hbm.at[1], s2_v)

        def _vmax(ref):
            def body(i, m):
                return jnp.maximum(m, ref[pl.ds(i * 16, 16)])
            m = lax.fori_loop(0, N_NODES // 16, body,
                              jnp.full((16,), -jnp.inf, jnp.float32))
            r = m[0]
            for i in range(1, 16):
                r = jnp.maximum(r, m[i])
            return r

        mb = _vmax(s1_v) + _vmax(s2_v)
        c_shift = jnp.where(mb >= 0, mb, ALPHA * mb)

        def sup_body(u, psum):
            base = ebase + u * SUPER
            pltpu.sync_copy(esrc_hbm.at[pl.ds(base, SUPER)], sidx_v)
            pltpu.sync_copy(edst_hbm.at[pl.ds(base, SUPER)], didx_v)

            def grp(g, acc_p):
                si = sidx_v[pl.ds(g * 16, 16)]
                di = didx_v[pl.ds(g * 16, 16)]
                e = (plsc.load_gather(s1_v, [si])
                     + plsc.load_gather(s2_v, [di]))
                e = jnp.where(e >= 0, e, ALPHA * e)
                p = jnp.exp(e - c_shift)
                pb_v[pl.ds(g * 16, 16)] = p
                return acc_p + p

            psum = lax.fori_loop(0, SUPER // 16, grp, psum)
            pltpu.sync_copy(pb_v, p_hbm.at[pl.ds(base, SUPER)])
            return psum

        psum = lax.fori_loop(0, NSUP, sup_body,
                             jnp.zeros((16,), jnp.float32))
        psum_v[...] = psum
        pltpu.sync_copy(psum_v, psums_hbm.at[wid, 0])

    return weights_kernel


def _make_scatter_kernel(mesh):
    @functools.partial(
        pl.kernel,
        out_type=[
            jax.ShapeDtypeStruct((NC, N_NODES, OUT_F), jnp.float32),
        ],
        mesh=mesh,
        scratch_types=[
            pltpu.VMEM((CHUNK,), jnp.int32),
            pltpu.VMEM((CHUNK,), jnp.int32),
            pltpu.VMEM((CHUNK,), jnp.float32),
            pltpu.VMEM((CHUNK,), jnp.int32),
            pltpu.VMEM((CHUNK,), jnp.int32),
            pltpu.VMEM((CHUNK,), jnp.float32),
            pltpu.VMEM((CHUNK, OUT_F), jnp.float32),
            pltpu.VMEM((CHUNK, OUT_F), jnp.float32),
            pltpu.VMEM_SHARED((N_NODES, OUT_F), jnp.float32),
            pltpu.SemaphoreType.DMA,
            pltpu.SemaphoreType.DMA,
            pltpu.SemaphoreType.DMA,
            pltpu.SemaphoreType.DMA,
        ],
        compiler_params=pltpu.CompilerParams(needs_layout_passes=False),
    )
    def scatter_kernel(wh_hbm, esrc_hbm, edst_hbm, p_hbm, parts_hbm,
                       sidx0_v, didx0_v, p0_v, sidx1_v, didx1_v, p1_v,
                       rows0_v, rows1_v, acc,
                       semi0, semi1, semg0, semg1):
        cid = lax.axis_index("c")
        sid = lax.axis_index("s")
        wid = sid * NC + cid
        ebase = wid * EPW

        zero16 = jnp.zeros((16,), jnp.float32)

        def zrow(i, carry):
            for j in range(GROUPS):
                rows0_v[i, pl.ds(j * 16, 16)] = zero16
            return carry

        lax.fori_loop(0, CHUNK, zrow, 0)
        zbase = sid * RSTRIPE
        for kk in range(RSTRIPE // CHUNK):
            pltpu.sync_copy(rows0_v,
                            acc.at[pl.ds(zbase + kk * CHUNK, CHUNK)])
        if RSTRIPE % CHUNK:
            pltpu.sync_copy(
                rows0_v.at[pl.ds(0, RSTRIPE % CHUNK)],
                acc.at[pl.ds(zbase + (RSTRIPE // CHUNK) * CHUNK,
                             RSTRIPE % CHUNK)])

        @pl.when(sid == NS - 1)
        def _zero_tail():
            pltpu.sync_copy(rows0_v.at[pl.ds(0, RTAIL)],
                            acc.at[pl.ds(NS * RSTRIPE, RTAIL)])

        plsc.subcore_barrier()

        def load_chunk(c, sidx, didx, pv, semi):
            base = ebase + c * CHUNK
            pltpu.async_copy(esrc_hbm.at[pl.ds(base, CHUNK)], sidx, semi)
            pltpu.async_copy(edst_hbm.at[pl.ds(base, CHUNK)], didx, semi)
            pltpu.async_copy(p_hbm.at[pl.ds(base, CHUNK)], pv, semi)

        def wait_chunk(c, sidx, didx, pv, semi):
            base = ebase + c * CHUNK
            pltpu.make_async_copy(
                esrc_hbm.at[pl.ds(base, CHUNK)], sidx, semi).wait()
            pltpu.make_async_copy(
                edst_hbm.at[pl.ds(base, CHUNK)], didx, semi).wait()
            pltpu.make_async_copy(
                p_hbm.at[pl.ds(base, CHUNK)], pv, semi).wait()

        def scale(rows, pv):
            def scale_group(g, carry):
                p16 = pv[pl.ds(g * 16, 16)]
                for j in range(16):
                    pe = p16[j]
                    ei = g * 16 + j
                    for k in range(GROUPS):
                        sl = pl.ds(k * 16, 16)
                        rows[ei, sl] = rows[ei, sl] * pe
                return carry
            lax.fori_loop(0, GROUPS, scale_group, 0)

        load_chunk(0, sidx0_v, didx0_v, p0_v, semi0)
        load_chunk(1, sidx1_v, didx1_v, p1_v, semi1)

        def pair_body(j, carry):
            c0 = 2 * j
            c1 = c0 + 1
            wait_chunk(c0, sidx0_v, didx0_v, p0_v, semi0)
            pltpu.async_copy(wh_hbm.at[didx0_v], rows0_v, semg0)
            wait_chunk(c1, sidx1_v, didx1_v, p1_v, semi1)
            pltpu.async_copy(wh_hbm.at[didx1_v], rows1_v, semg1)
            pltpu.make_async_copy(wh_hbm.at[didx0_v], rows0_v, semg0).wait()
            scale(rows0_v, p0_v)
            pltpu.sync_copy(rows0_v, acc.at[sidx0_v], add=True)
            load_chunk(c0 + 2, sidx0_v, didx0_v, p0_v, semi0)
            pltpu.make_async_copy(wh_hbm.at[didx1_v], rows1_v, semg1).wait()
            scale(rows1_v, p1_v)
            pltpu.sync_copy(rows1_v, acc.at[sidx1_v], add=True)
            load_chunk(c1 + 2, sidx1_v, didx1_v, p1_v, semi1)
            return carry

        lax.fori_loop(0, NFULL // 2 - 1, pair_body, 0)

        c0 = NFULL - 2
        c1 = NFULL - 1
        wait_chunk(c0, sidx0_v, didx0_v, p0_v, semi0)
        pltpu.async_copy(wh_hbm.at[didx0_v], rows0_v, semg0)
        wait_chunk(c1, sidx1_v, didx1_v, p1_v, semi1)
        pltpu.async_copy(wh_hbm.at[didx1_v], rows1_v, semg1)
        pltpu.make_async_copy(wh_hbm.at[didx0_v], rows0_v, semg0).wait()
        scale(rows0_v, p0_v)
        pltpu.sync_copy(rows0_v, acc.at[sidx0_v], add=True)
        pltpu.make_async_copy(wh_hbm.at[didx1_v], rows1_v, semg1).wait()
        scale(rows1_v, p1_v)
        pltpu.sync_copy(rows1_v, acc.at[sidx1_v], add=True)

        tbase = ebase + NFULL * CHUNK
        pltpu.sync_copy(esrc_hbm.at[pl.ds(tbase, TAIL)],
                        sidx0_v.at[pl.ds(0, TAIL)])
        pltpu.sync_copy(edst_hbm.at[pl.ds(tbase, TAIL)],
                        didx0_v.at[pl.ds(0, TAIL)])
        pltpu.sync_copy(p_hbm.at[pl.ds(tbase, TAIL)],
                        p0_v.at[pl.ds(0, TAIL)])
        for g in range(1, GROUPS):
            p0_v[pl.ds(g * 16, 16)] = zero16
        pltpu.async_copy(wh_hbm.at[didx0_v], rows0_v, semg0).wait()
        scale(rows0_v, p0_v)
        pltpu.sync_copy(rows0_v, acc.at[sidx0_v], add=True)

        plsc.subcore_barrier()

        pltpu.sync_copy(acc.at[pl.ds(sid * RSTRIPE, RSTRIPE)],
                        parts_hbm.at[cid, pl.ds(sid * RSTRIPE, RSTRIPE)])

        @pl.when(sid == NS - 1)
        def _out_tail():
            pltpu.sync_copy(acc.at[pl.ds(NS * RSTRIPE, RTAIL)],
                            parts_hbm.at[cid, pl.ds(NS * RSTRIPE, RTAIL)])

    return scatter_kernel


_sc_cache = None


def _get_sc_kernels():
    global _sc_cache
    if _sc_cache is None:
        mesh = plsc.VectorSubcoreMesh(core_axis_name="c",
                                      subcore_axis_name="s",
                                      num_cores=NC, num_subcores=NS)
        _sc_cache = (_make_weights_kernel(mesh), _make_scatter_kernel(mesh))
    return _sc_cache


def _ep_body(parts_ref, psums_ref, o_ref):
    s = jnp.sum(psums_ref[...])
    v = (parts_ref[0] + parts_ref[1]) * (1.0 / s)
    o_ref[...] = jnp.where(v > 0, v, jnp.exp(v) - 1.0)


def _epilogue(parts, psums):
    return pl.pallas_call(
        _ep_body,
        grid=(N_NODES // BLK,),
        in_specs=[
            pl.BlockSpec((NC, BLK, OUT_F), lambda i: (0, i, 0)),
            pl.BlockSpec((NW, 1, 16), lambda i: (0, 0, 0)),
        ],
        out_specs=pl.BlockSpec((BLK, OUT_F), lambda i: (i, 0)),
        out_shape=jax.ShapeDtypeStruct((N_NODES, OUT_F), jnp.float32),
    )(parts, psums)


def kernel(x, edge_index, W, a):
    x = x.astype(jnp.float32)
    ei = edge_index.astype(jnp.int32)
    a_col = a[:, 0].astype(jnp.float32)
    At = jnp.zeros((IN_F, IN_F), jnp.float32)
    At = At.at[0, :].set(a_col[:IN_F]).at[1, :].set(a_col[IN_F:])
    Wh, s_all = _matmul_scores(x, W.astype(jnp.float32), At)
    weights_k, scatter_k = _get_sc_kernels()
    p_edges, psums = weights_k(s_all, ei[0], ei[1])
    (parts,) = scatter_k(Wh, ei[0], ei[1], p_edges)
    return _epilogue(parts, psums)


# speedup vs baseline: 12.9415x; 1.1332x over previous
import functools

import jax
import jax.numpy as jnp
from jax import lax
from jax.experimental import pallas as pl
from jax.experimental.pallas import tpu as pltpu
from jax.experimental.pallas import tpu_sc as plsc

IN_F = 128
OUT_F = 128
ALPHA = 0.2
N_NODES = 10000
N_EDGES = 320000

NC = 2
NS = 16
NW = NC * NS
EPW = N_EDGES // NW
CHUNK = 128
NFULL = EPW // CHUNK
TAIL = EPW - NFULL * CHUNK
GROUPS = CHUNK // 16
SUPER = 2000
NSUP = EPW // SUPER
RSTRIPE = 624
RTAIL = N_NODES - NS * RSTRIPE

BLK = 1000


def _mm_body(x_ref, w_ref, at_ref, wh_ref, s_ref):
    wh = jnp.dot(x_ref[...], w_ref[...], preferred_element_type=jnp.float32)
    wh_ref[...] = wh
    s_ref[...] = lax.dot_general(
        at_ref[...], wh, (((1,), (1,)), ((), ())),
        preferred_element_type=jnp.float32)


def _matmul_scores(x, W, At):
    return pl.pallas_call(
        _mm_body,
        out_shape=[
            jax.ShapeDtypeStruct((N_NODES, OUT_F), jnp.float32),
            jax.ShapeDtypeStruct((IN_F, N_NODES), jnp.float32),
        ],
    )(x, W, At)


def _make_weights_kernel(mesh):
    @functools.partial(
        pl.kernel,
        out_type=[
            jax.ShapeDtypeStruct((N_EDGES,), jnp.float32),
            jax.ShapeDtypeStruct((NW, 1, 16), jnp.float32),
        ],
        mesh=mesh,
        scratch_types=[
            pltpu.VMEM((N_NODES,), jnp.float32),
            pltpu.VMEM((N_NODES,), jnp.float32),
            pltpu.VMEM((SUPER,), jnp.int32),
            pltpu.VMEM((SUPER,), jnp.int32),
            pltpu.VMEM((SUPER,), jnp.float32),
            pltpu.VMEM((16,), jnp.float32),
        ],
        compiler_params=pltpu.CompilerParams(needs_layout_passes=False),
    )
    def weights_kernel(s_hbm, esrc_hbm, edst_hbm, p_hbm, psums_hbm,
                       s1_v, s2_v, sidx_v, didx_v, pb_v, psum_v):
        cid = lax.axis_index("c")
        sid = lax.axis_index("s")
        wid = sid * NC + cid
        ebase = wid * EPW

        pltpu.sync_copy(s_hbm.at[0], s1_v)
        pltpu.sync_copy(s_hbm.at[1], s2_v)

        def _vmax(ref):
            def body(i, m):
                return jnp.maximum(m, ref[pl.ds(i * 16, 16)])
            m = lax.fori_loop(0, N_NODES // 16, body,
                              jnp.full((16,), -jnp.inf, jnp.float32))
            r = m[0]
            for i in range(1, 16):
                r = jnp.maximum(r, m[i])
            return r

        mb = _vmax(s1_v) + _vmax(s2_v)
        c_shift = jnp.where(mb >= 0, mb, ALPHA * mb)

        def sup_body(u, psum):
            base = ebase + u * SUPER
            pltpu.sync_copy(esrc_hbm.at[pl.ds(base, SUPER)], sidx_v)
            pltpu.sync_copy(edst_hbm.at[pl.ds(base, SUPER)], didx_v)

            def grp(g, acc_p):
                si = sidx_v[pl.ds(g * 16, 16)]
                di = didx_v[pl.ds(g * 16, 16)]
                e = (plsc.load_gather(s1_v, [si])
                     + plsc.load_gather(s2_v, [di]))
                e = jnp.where(e >= 0, e, ALPHA * e)
                p = jnp.exp(e - c_shift)
                pb_v[pl.ds(g * 16, 16)] = p
                return acc_p + p

            psum = lax.fori_loop(0, SUPER // 16, grp, psum)
            pltpu.sync_copy(pb_v, p_hbm.at[pl.ds(base, SUPER)])
            return psum

        psum = lax.fori_loop(0, NSUP, sup_body,
                             jnp.zeros((16,), jnp.float32))
        psum_v[...] = psum
        pltpu.sync_copy(psum_v, psums_hbm.at[wid, 0])

    return weights_kernel


def _make_scatter_kernel(mesh):
    @functools.partial(
        pl.kernel,
        out_type=[
            jax.ShapeDtypeStruct((NC, N_NODES, OUT_F), jnp.float32),
        ],
        mesh=mesh,
        scratch_types=[
            pltpu.VMEM((CHUNK,), jnp.int32),
            pltpu.VMEM((CHUNK,), jnp.int32),
            pltpu.VMEM((CHUNK,), jnp.float32),
            pltpu.VMEM((CHUNK,), jnp.int32),
            pltpu.VMEM((CHUNK,), jnp.int32),
            pltpu.VMEM((CHUNK,), jnp.float32),
            pltpu.VMEM((CHUNK, OUT_F), jnp.float32),
            pltpu.VMEM((CHUNK, OUT_F), jnp.float32),
            pltpu.VMEM((CHUNK,), jnp.int32),
            pltpu.VMEM((CHUNK,), jnp.int32),
            pltpu.VMEM_SHARED((N_NODES, OUT_F), jnp.float32),
            pltpu.SemaphoreType.DMA,
            pltpu.SemaphoreType.DMA,
            pltpu.SemaphoreType.DMA,
            pltpu.SemaphoreType.DMA,
            pltpu.SemaphoreType.DMA,
            pltpu.SemaphoreType.DMA,
        ],
        compiler_params=pltpu.CompilerParams(needs_layout_passes=False),
    )
    def scatter_kernel(wh_hbm, esrc_hbm, edst_hbm, p_hbm, parts_hbm,
                       sidx0_v, didx0_v, p0_v, sidx1_v, didx1_v, p1_v,
                       rows0_v, rows1_v, ssidx0_v, ssidx1_v, acc,
                       semi0, semi1, semg0, semg1, sems0, sems1):
        cid = lax.axis_index("c")
        sid = lax.axis_index("s")
        wid = sid * NC + cid
        ebase = wid * EPW

        zero16 = jnp.zeros((16,), jnp.float32)

        def zrow(i, carry):
            for j in range(GROUPS):
                rows0_v[i, pl.ds(j * 16, 16)] = zero16
            return carry

        lax.fori_loop(0, CHUNK, zrow, 0)
        zbase = sid * RSTRIPE
        for kk in range(RSTRIPE // CHUNK):
            pltpu.sync_copy(rows0_v,
                            acc.at[pl.ds(zbase + kk * CHUNK, CHUNK)])
        if RSTRIPE % CHUNK:
            pltpu.sync_copy(
                rows0_v.at[pl.ds(0, RSTRIPE % CHUNK)],
                acc.at[pl.ds(zbase + (RSTRIPE // CHUNK) * CHUNK,
                             RSTRIPE % CHUNK)])

        @pl.when(sid == NS - 1)
        def _zero_tail():
            pltpu.sync_copy(rows0_v.at[pl.ds(0, RTAIL)],
                            acc.at[pl.ds(NS * RSTRIPE, RTAIL)])

        plsc.subcore_barrier()

        def load_chunk(c, sidx, didx, pv, semi):
            base = ebase + c * CHUNK
            pltpu.async_copy(esrc_hbm.at[pl.ds(base, CHUNK)], sidx, semi)
            pltpu.async_copy(edst_hbm.at[pl.ds(base, CHUNK)], didx, semi)
            pltpu.async_copy(p_hbm.at[pl.ds(base, CHUNK)], pv, semi)

        def wait_chunk(c, sidx, didx, pv, semi):
            base = ebase + c * CHUNK
            pltpu.make_async_copy(
                esrc_hbm.at[pl.ds(base, CHUNK)], sidx, semi).wait()
            pltpu.make_async_copy(
                edst_hbm.at[pl.ds(base, CHUNK)], didx, semi).wait()
            pltpu.make_async_copy(
                p_hbm.at[pl.ds(base, CHUNK)], pv, semi).wait()

        def scale(rows, pv):
            def scale_group(g, carry):
                p16 = pv[pl.ds(g * 16, 16)]
                for j in range(16):
                    pe = p16[j]
                    ei = g * 16 + j
                    for k in range(GROUPS):
                        sl = pl.ds(k * 16, 16)
                        rows[ei, sl] = rows[ei, sl] * pe
                return carry
            lax.fori_loop(0, GROUPS, scale_group, 0)

        def copy_idx(src, dst):
            for g in range(GROUPS):
                dst[pl.ds(g * 16, 16)] = src[pl.ds(g * 16, 16)]

        def drain_scatter(rows, ssidx, sems):
            pltpu.make_async_copy(rows, acc.at[ssidx], sems).wait()

        load_chunk(0, sidx0_v, didx0_v, p0_v, semi0)
        load_chunk(1, sidx1_v, didx1_v, p1_v, semi1)

        def pair(j, drain, load_next):
            c0 = 2 * j
            c1 = c0 + 1
            wait_chunk(c0, sidx0_v, didx0_v, p0_v, semi0)
            if drain:
                drain_scatter(rows0_v, ssidx0_v, sems0)
            pltpu.async_copy(wh_hbm.at[didx0_v], rows0_v, semg0)
            wait_chunk(c1, sidx1_v, didx1_v, p1_v, semi1)
            if drain:
                drain_scatter(rows1_v, ssidx1_v, sems1)
            pltpu.async_copy(wh_hbm.at[didx1_v], rows1_v, semg1)
            pltpu.make_async_copy(wh_hbm.at[didx0_v], rows0_v, semg0).wait()
            scale(rows0_v, p0_v)
            copy_idx(sidx0_v, ssidx0_v)
            pltpu.async_copy(rows0_v, acc.at[ssidx0_v], sems0, add=True)
            if load_next:
                load_chunk(c0 + 2, sidx0_v, didx0_v, p0_v, semi0)
            pltpu.make_async_copy(wh_hbm.at[didx1_v], rows1_v, semg1).wait()
            scale(rows1_v, p1_v)
            copy_idx(sidx1_v, ssidx1_v)
            pltpu.async_copy(rows1_v, acc.at[ssidx1_v], sems1, add=True)
            if load_next:
                load_chunk(c1 + 2, sidx1_v, didx1_v, p1_v, semi1)

        pair(0, drain=False, load_next=True)
        lax.fori_loop(1, NFULL // 2 - 1,
                      lambda j, c: (pair(j, drain=True, load_next=True), c)[1],
                      0)
        pair(NFULL // 2 - 1, drain=True, load_next=False)

        tbase = ebase + NFULL * CHUNK
        pltpu.sync_copy(esrc_hbm.at[pl.ds(tbase, TAIL)],
                        sidx0_v.at[pl.ds(0, TAIL)])
        pltpu.sync_copy(edst_hbm.at[pl.ds(tbase, TAIL)],
                        didx0_v.at[pl.ds(0, TAIL)])
        pltpu.sync_copy(p_hbm.at[pl.ds(tbase, TAIL)],
                        p0_v.at[pl.ds(0, TAIL)])
        for g in range(1, GROUPS):
            p0_v[pl.ds(g * 16, 16)] = zero16
        drain_scatter(rows0_v, ssidx0_v, sems0)
        pltpu.async_copy(wh_hbm.at[didx0_v], rows0_v, semg0).wait()
        scale(rows0_v, p0_v)
        pltpu.sync_copy(rows0_v, acc.at[sidx0_v], add=True)
        drain_scatter(rows1_v, ssidx1_v, sems1)

        plsc.subcore_barrier()

        pltpu.sync_copy(acc.at[pl.ds(sid * RSTRIPE, RSTRIPE)],
                        parts_hbm.at[cid, pl.ds(sid * RSTRIPE, RSTRIPE)])

        @pl.when(sid == NS - 1)
        def _out_tail():
            pltpu.sync_copy(acc.at[pl.ds(NS * RSTRIPE, RTAIL)],
                            parts_hbm.at[cid, pl.ds(NS * RSTRIPE, RTAIL)])

    return scatter_kernel


_sc_cache = None


def _get_sc_kernels():
    global _sc_cache
    if _sc_cache is None:
        mesh = plsc.VectorSubcoreMesh(core_axis_name="c",
                                      subcore_axis_name="s",
                                      num_cores=NC, num_subcores=NS)
        _sc_cache = (_make_weights_kernel(mesh), _make_scatter_kernel(mesh))
    return _sc_cache


def _ep_body(parts_ref, psums_ref, o_ref):
    s = jnp.sum(psums_ref[...])
    v = (parts_ref[0] + parts_ref[1]) * (1.0 / s)
    o_ref[...] = jnp.where(v > 0, v, jnp.exp(v) - 1.0)


def _epilogue(parts, psums):
    return pl.pallas_call(
        _ep_body,
        grid=(N_NODES // BLK,),
        in_specs=[
            pl.BlockSpec((NC, BLK, OUT_F), lambda i: (0, i, 0)),
            pl.BlockSpec((NW, 1, 16), lambda i: (0, 0, 0)),
        ],
        out_specs=pl.BlockSpec((BLK, OUT_F), lambda i: (i, 0)),
        out_shape=jax.ShapeDtypeStruct((N_NODES, OUT_F), jnp.float32),
    )(parts, psums)


def kernel(x, edge_index, W, a):
    x = x.astype(jnp.float32)
    ei = edge_index.astype(jnp.int32)
    a_col = a[:, 0].astype(jnp.float32)
    At = jnp.zeros((IN_F, IN_F), jnp.float32)
    At = At.at[0, :].set(a_col[:IN_F]).at[1, :].set(a_col[IN_F:])
    Wh, s_all = _matmul_scores(x, W.astype(jnp.float32), At)
    weights_k, scatter_k = _get_sc_kernels()
    p_edges, psums = weights_k(s_all, ei[0], ei[1])
    (parts,) = scatter_k(Wh, ei[0], ei[1], p_edges)
    return _epilogue(parts, psums)


# speedup vs baseline: 14.2560x; 1.1016x over previous
import functools

import jax
import jax.numpy as jnp
from jax import lax
from jax.experimental import pallas as pl
from jax.experimental.pallas import tpu as pltpu
from jax.experimental.pallas import tpu_sc as plsc

IN_F = 128
OUT_F = 128
ALPHA = 0.2
N_NODES = 10000
N_EDGES = 320000

NC = 2
NS = 16
NW = NC * NS
EPW = N_EDGES // NW
CHUNK = 128
NFULL = EPW // CHUNK
TAIL = EPW - NFULL * CHUNK
GROUPS = CHUNK // 16
RSTRIPE = 624
RTAIL = N_NODES - NS * RSTRIPE

BLK = 1000


def _mm_body(x_ref, w_ref, at_ref, wh_ref, s_ref, c_ref):
    wh = jnp.dot(x_ref[...], w_ref[...], preferred_element_type=jnp.float32)
    wh_ref[...] = wh
    s = lax.dot_general(at_ref[...], wh, (((1,), (1,)), ((), ())),
                        preferred_element_type=jnp.float32)
    s_ref[...] = s
    m = jnp.max(s[0]) + jnp.max(s[1])
    c = jnp.where(m >= 0, m, ALPHA * m)
    c_ref[...] = jnp.full((8, 128), c, jnp.float32)


def _matmul_scores(x, W, At):
    return pl.pallas_call(
        _mm_body,
        out_shape=[
            jax.ShapeDtypeStruct((N_NODES, OUT_F), jnp.float32),
            jax.ShapeDtypeStruct((IN_F, N_NODES), jnp.float32),
            jax.ShapeDtypeStruct((8, 128), jnp.float32),
        ],
    )(x, W, At)


def _make_sc_kernel():
    mesh = plsc.VectorSubcoreMesh(core_axis_name="c", subcore_axis_name="s",
                                  num_cores=NC, num_subcores=NS)

    @functools.partial(
        pl.kernel,
        out_type=[
            jax.ShapeDtypeStruct((NC, N_NODES, OUT_F), jnp.float32),
            jax.ShapeDtypeStruct((NW, 1, 16), jnp.float32),
        ],
        mesh=mesh,
        scratch_types=[
            pltpu.VMEM((CHUNK,), jnp.int32),
            pltpu.VMEM((CHUNK,), jnp.int32),
            pltpu.VMEM((CHUNK,), jnp.int32),
            pltpu.VMEM((CHUNK,), jnp.int32),
            pltpu.VMEM((CHUNK,), jnp.float32),
            pltpu.VMEM((CHUNK,), jnp.float32),
            pltpu.VMEM((CHUNK,), jnp.float32),
            pltpu.VMEM((CHUNK,), jnp.float32),
            pltpu.VMEM((CHUNK, OUT_F), jnp.float32),
            pltpu.VMEM((CHUNK, OUT_F), jnp.float32),
            pltpu.VMEM((CHUNK,), jnp.int32),
            pltpu.VMEM((CHUNK,), jnp.int32),
            pltpu.VMEM((16,), jnp.float32),
            pltpu.VMEM((16,), jnp.float32),
            pltpu.VMEM_SHARED((N_NODES, OUT_F), jnp.float32),
            pltpu.SemaphoreType.DMA,
            pltpu.SemaphoreType.DMA,
            pltpu.SemaphoreType.DMA,
            pltpu.SemaphoreType.DMA,
            pltpu.SemaphoreType.DMA,
            pltpu.SemaphoreType.DMA,
        ],
        compiler_params=pltpu.CompilerParams(needs_layout_passes=False),
    )
    def sc_kernel(wh_hbm, s1_hbm, s2_hbm, esrc_hbm, edst_hbm, c_hbm,
                  parts_hbm, psums_hbm,
                  sidx0_v, didx0_v, sidx1_v, didx1_v,
                  sv1_0, sv2_0, sv1_1, sv2_1,
                  rows0_v, rows1_v, ssidx0_v, ssidx1_v,
                  cbuf_v, psum_v, acc,
                  semi0, semi1, semg0, semg1, sems0, sems1):
        cid = lax.axis_index("c")
        sid = lax.axis_index("s")
        wid = sid * NC + cid
        ebase = wid * EPW

        zero16 = jnp.zeros((16,), jnp.float32)

        pltpu.sync_copy(c_hbm.at[0, pl.ds(0, 16)], cbuf_v)
        c_shift = cbuf_v[pl.ds(0, 16)][0]

        def zrow(i, carry):
            for j in range(GROUPS):
                rows0_v[i, pl.ds(j * 16, 16)] = zero16
            return carry

        lax.fori_loop(0, CHUNK, zrow, 0)
        zbase = sid * RSTRIPE
        for kk in range(RSTRIPE // CHUNK):
            pltpu.sync_copy(rows0_v,
                            acc.at[pl.ds(zbase + kk * CHUNK, CHUNK)])
        if RSTRIPE % CHUNK:
            pltpu.sync_copy(
                rows0_v.at[pl.ds(0, RSTRIPE % CHUNK)],
                acc.at[pl.ds(zbase + (RSTRIPE // CHUNK) * CHUNK,
                             RSTRIPE % CHUNK)])

        @pl.when(sid == NS - 1)
        def _zero_tail():
            pltpu.sync_copy(rows0_v.at[pl.ds(0, RTAIL)],
                            acc.at[pl.ds(NS * RSTRIPE, RTAIL)])

        plsc.subcore_barrier()

        def load_idx(c, sidx, didx, semi):
            base = ebase + c * CHUNK
            pltpu.async_copy(esrc_hbm.at[pl.ds(base, CHUNK)], sidx, semi)
            pltpu.async_copy(edst_hbm.at[pl.ds(base, CHUNK)], didx, semi)

        def wait_idx(c, sidx, didx, semi):
            base = ebase + c * CHUNK
            pltpu.make_async_copy(
                esrc_hbm.at[pl.ds(base, CHUNK)], sidx, semi).wait()
            pltpu.make_async_copy(
                edst_hbm.at[pl.ds(base, CHUNK)], didx, semi).wait()

        def issue_gathers(sidx, didx, sv1, sv2, rows, semg):
            pltpu.async_copy(wh_hbm.at[didx], rows, semg)
            pltpu.async_copy(s1_hbm.at[sidx], sv1, semg)
            pltpu.async_copy(s2_hbm.at[didx], sv2, semg)

        def wait_gathers(sidx, didx, sv1, sv2, rows, semg):
            pltpu.make_async_copy(wh_hbm.at[didx], rows, semg).wait()
            pltpu.make_async_copy(s1_hbm.at[sidx], sv1, semg).wait()
            pltpu.make_async_copy(s2_hbm.at[didx], sv2, semg).wait()

        def weigh_scale(sv1, sv2, rows, psum):
            def group(g, acc_p):
                e = sv1[pl.ds(g * 16, 16)] + sv2[pl.ds(g * 16, 16)]
                e = jnp.where(e >= 0, e, ALPHA * e)
                p16 = jnp.exp(e - c_shift)
                for j in range(16):
                    pe = p16[j]
                    ei = g * 16 + j
                    for k in range(GROUPS):
                        sl = pl.ds(k * 16, 16)
                        rows[ei, sl] = rows[ei, sl] * pe
                return acc_p + p16
            return lax.fori_loop(0, GROUPS, group, psum)

        def copy_idx(src, dst):
            for g in range(GROUPS):
                dst[pl.ds(g * 16, 16)] = src[pl.ds(g * 16, 16)]

        def drain_scatter(rows, ssidx, sems):
            pltpu.make_async_copy(rows, acc.at[ssidx], sems).wait()

        load_idx(0, sidx0_v, didx0_v, semi0)
        load_idx(1, sidx1_v, didx1_v, semi1)

        def pair(j, psum, drain, load_next):
            c0 = 2 * j
            c1 = c0 + 1
            wait_idx(c0, sidx0_v, didx0_v, semi0)
            if drain:
                drain_scatter(rows0_v, ssidx0_v, sems0)
            issue_gathers(sidx0_v, didx0_v, sv1_0, sv2_0, rows0_v, semg0)
            wait_idx(c1, sidx1_v, didx1_v, semi1)
            if drain:
                drain_scatter(rows1_v, ssidx1_v, sems1)
            issue_gathers(sidx1_v, didx1_v, sv1_1, sv2_1, rows1_v, semg1)
            wait_gathers(sidx0_v, didx0_v, sv1_0, sv2_0, rows0_v, semg0)
            psum = weigh_scale(sv1_0, sv2_0, rows0_v, psum)
            copy_idx(sidx0_v, ssidx0_v)
            pltpu.async_copy(rows0_v, acc.at[ssidx0_v], sems0, add=True)
            if load_next:
                load_idx(c0 + 2, sidx0_v, didx0_v, semi0)
            wait_gathers(sidx1_v, didx1_v, sv1_1, sv2_1, rows1_v, semg1)
            psum = weigh_scale(sv1_1, sv2_1, rows1_v, psum)
            copy_idx(sidx1_v, ssidx1_v)
            pltpu.async_copy(rows1_v, acc.at[ssidx1_v], sems1, add=True)
            if load_next:
                load_idx(c1 + 2, sidx1_v, didx1_v, semi1)
            return psum

        psum = pair(0, jnp.zeros((16,), jnp.float32),
                    drain=False, load_next=True)
        psum = lax.fori_loop(
            1, NFULL // 2 - 1,
            lambda j, ps: pair(j, ps, drain=True, load_next=True), psum)
        psum = pair(NFULL // 2 - 1, psum, drain=True, load_next=False)

        tbase = ebase + NFULL * CHUNK
        pltpu.sync_copy(esrc_hbm.at[pl.ds(tbase, TAIL)],
                        sidx0_v.at[pl.ds(0, TAIL)])
        pltpu.sync_copy(edst_hbm.at[pl.ds(tbase, TAIL)],
                        didx0_v.at[pl.ds(0, TAIL)])
        drain_scatter(rows0_v, ssidx0_v, sems0)
        pltpu.async_copy(wh_hbm.at[didx0_v], rows0_v, semg0)
        pltpu.async_copy(s1_hbm.at[sidx0_v], sv1_0, semg0)
        pltpu.async_copy(s2_hbm.at[didx0_v], sv2_0, semg0)
        pltpu.make_async_copy(wh_hbm.at[didx0_v], rows0_v, semg0).wait()
        pltpu.make_async_copy(s1_hbm.at[sidx0_v], sv1_0, semg0).wait()
        pltpu.make_async_copy(s2_hbm.at[didx0_v], sv2_0, semg0).wait()
        e = sv1_0[pl.ds(0, 16)] + sv2_0[pl.ds(0, 16)]
        e = jnp.where(e >= 0, e, ALPHA * e)
        tp = jnp.exp(e - c_shift)
        psum = psum + tp
        for j in range(16):
            pe = tp[j]
            for k in range(GROUPS):
                sl = pl.ds(k * 16, 16)
                rows0_v[j, sl] = rows0_v[j, sl] * pe

        def zero_rest(i, carry):
            for k in range(GROUPS):
                rows0_v[i, pl.ds(k * 16, 16)] = zero16
            return carry

        lax.fori_loop(TAIL, CHUNK, zero_rest, 0)
        pltpu.sync_copy(rows0_v, acc.at[sidx0_v], add=True)
        drain_scatter(rows1_v, ssidx1_v, sems1)

        plsc.subcore_barrier()

        pltpu.sync_copy(acc.at[pl.ds(sid * RSTRIPE, RSTRIPE)],
                        parts_hbm.at[cid, pl.ds(sid * RSTRIPE, RSTRIPE)])

        @pl.when(sid == NS - 1)
        def _out_tail():
            pltpu.sync_copy(acc.at[pl.ds(NS * RSTRIPE, RTAIL)],
                            parts_hbm.at[cid, pl.ds(NS * RSTRIPE, RTAIL)])

        psum_v[...] = psum
        pltpu.sync_copy(psum_v, psums_hbm.at[wid, 0])

    return sc_kernel


_sc_cache = None


def _get_sc_kernel():
    global _sc_cache
    if _sc_cache is None:
        _sc_cache = _make_sc_kernel()
    return _sc_cache


def _ep_body(parts_ref, psums_ref, o_ref):
    s = jnp.sum(psums_ref[...])
    v = (parts_ref[0] + parts_ref[1]) * (1.0 / s)
    o_ref[...] = jnp.where(v > 0, v, jnp.exp(v) - 1.0)


def _epilogue(parts, psums):
    return pl.pallas_call(
        _ep_body,
        grid=(N_NODES // BLK,),
        in_specs=[
            pl.BlockSpec((NC, BLK, OUT_F), lambda i: (0, i, 0)),
            pl.BlockSpec((NW, 1, 16), lambda i: (0, 0, 0)),
        ],
        out_specs=pl.BlockSpec((BLK, OUT_F), lambda i: (i, 0)),
        out_shape=jax.ShapeDtypeStruct((N_NODES, OUT_F), jnp.float32),
    )(parts, psums)


def kernel(x, edge_index, W, a):
    x = x.astype(jnp.float32)
    ei = edge_index.astype(jnp.int32)
    a_col = a[:, 0].astype(jnp.float32)
    At = jnp.zeros((IN_F, IN_F), jnp.float32)
    At = At.at[0, :].set(a_col[:IN_F]).at[1, :].set(a_col[IN_F:])
    Wh, s_all, c_arr = _matmul_scores(x, W.astype(jnp.float32), At)
    parts, psums = _get_sc_kernel()(Wh, s_all[0], s_all[1],
                                    ei[0], ei[1], c_arr)
    return _epilogue(parts, psums)


# speedup vs baseline: 16.3969x; 1.1502x over previous
import functools

import jax
import jax.numpy as jnp
from jax import lax
from jax.experimental import pallas as pl
from jax.experimental.pallas import tpu as pltpu
from jax.experimental.pallas import tpu_sc as plsc

IN_F = 128
OUT_F = 128
ALPHA = 0.2
N_NODES = 10000
N_EDGES = 320000

NC = 2
NS = 16
NW = NC * NS
EPW = N_EDGES // NW
CHUNK = 128
NFULL = EPW // CHUNK
TAIL = EPW - NFULL * CHUNK
GROUPS = CHUNK // 16
RSTRIPE = 624
RTAIL = N_NODES - NS * RSTRIPE

BLK = 1000


def _mm_body(x_ref, w_ref, at_ref, wh_ref, s_ref, c_ref):
    wh = jnp.dot(x_ref[...], w_ref[...], preferred_element_type=jnp.float32)
    wh_ref[...] = wh
    s = lax.dot_general(at_ref[...], wh, (((1,), (1,)), ((), ())),
                        preferred_element_type=jnp.float32)
    s_ref[...] = s
    m = jnp.max(s[0]) + jnp.max(s[1])
    c = jnp.where(m >= 0, m, ALPHA * m)
    c_ref[...] = jnp.full((8, 128), c, jnp.float32)


def _matmul_scores(x, W, At):
    return pl.pallas_call(
        _mm_body,
        out_shape=[
            jax.ShapeDtypeStruct((N_NODES, OUT_F), jnp.float32),
            jax.ShapeDtypeStruct((IN_F, N_NODES), jnp.float32),
            jax.ShapeDtypeStruct((8, 128), jnp.float32),
        ],
    )(x, W, At)


def _make_sc_kernel():
    mesh = plsc.VectorSubcoreMesh(core_axis_name="c", subcore_axis_name="s",
                                  num_cores=NC, num_subcores=NS)

    nbuf_scratch = []
    for _ in range(3):
        nbuf_scratch += [
            pltpu.VMEM((CHUNK,), jnp.int32),
            pltpu.VMEM((CHUNK,), jnp.int32),
            pltpu.VMEM((CHUNK,), jnp.int32),
            pltpu.VMEM((CHUNK,), jnp.float32),
            pltpu.VMEM((CHUNK, OUT_F), jnp.float32),
        ]

    @functools.partial(
        pl.kernel,
        out_type=[
            jax.ShapeDtypeStruct((NC, N_NODES, OUT_F), jnp.float32),
            jax.ShapeDtypeStruct((NW, 1, 16), jnp.float32),
        ],
        mesh=mesh,
        scratch_types=nbuf_scratch + [
            pltpu.VMEM_SHARED((N_NODES, OUT_F), jnp.float32),
            pltpu.SemaphoreType.DMA,
            pltpu.SemaphoreType.DMA,
            pltpu.SemaphoreType.DMA,
            pltpu.SemaphoreType.DMA,
            pltpu.SemaphoreType.DMA,
            pltpu.SemaphoreType.DMA,
            pltpu.SemaphoreType.DMA,
            pltpu.SemaphoreType.DMA,
            pltpu.SemaphoreType.DMA,
        ],
        compiler_params=pltpu.CompilerParams(needs_layout_passes=False),
    )
    def sc_kernel(wh_hbm, s1_hbm, s2_hbm, esrc_hbm, edst_hbm, c_hbm,
                  parts_hbm, psums_hbm,
                  sidx_a, didx_a, sv1_a, sv2_a, rows_a,
                  sidx_b, didx_b, sv1_b, sv2_b, rows_b,
                  sidx_c, didx_c, sv1_c, sv2_c, rows_c,
                  acc,
                  semi_a, semi_b, semi_c,
                  semg_a, semg_b, semg_c,
                  sems_a, sems_b, sems_c):
        SIDX = [sidx_a, sidx_b, sidx_c]
        DIDX = [didx_a, didx_b, didx_c]
        SV1 = [sv1_a, sv1_b, sv1_c]
        SV2 = [sv2_a, sv2_b, sv2_c]
        ROWS = [rows_a, rows_b, rows_c]
        SEMI = [semi_a, semi_b, semi_c]
        SEMG = [semg_a, semg_b, semg_c]
        SEMS = [sems_a, sems_b, sems_c]

        cid = lax.axis_index("c")
        sid = lax.axis_index("s")
        wid = sid * NC + cid
        ebase = wid * EPW

        zero16 = jnp.zeros((16,), jnp.float32)

        pltpu.sync_copy(c_hbm.at[0, pl.ds(0, 16)],
                        rows_a.at[0, pl.ds(0, 16)])
        c_shift = rows_a[0, pl.ds(0, 16)][0]

        def zrow(i, carry):
            for j in range(GROUPS):
                rows_a[i, pl.ds(j * 16, 16)] = zero16
            return carry

        lax.fori_loop(0, CHUNK, zrow, 0)
        zbase = sid * RSTRIPE
        for kk in range(RSTRIPE // CHUNK):
            pltpu.sync_copy(rows_a,
                            acc.at[pl.ds(zbase + kk * CHUNK, CHUNK)])
        if RSTRIPE % CHUNK:
            pltpu.sync_copy(
                rows_a.at[pl.ds(0, RSTRIPE % CHUNK)],
                acc.at[pl.ds(zbase + (RSTRIPE // CHUNK) * CHUNK,
                             RSTRIPE % CHUNK)])

        @pl.when(sid == NS - 1)
        def _zero_tail():
            pltpu.sync_copy(rows_a.at[pl.ds(0, RTAIL)],
                            acc.at[pl.ds(NS * RSTRIPE, RTAIL)])

        plsc.subcore_barrier()

        def load_idx(cc, b):
            base = ebase + cc * CHUNK
            pltpu.async_copy(esrc_hbm.at[pl.ds(base, CHUNK)], SIDX[b],
                             SEMI[b])
            pltpu.async_copy(edst_hbm.at[pl.ds(base, CHUNK)], DIDX[b],
                             SEMI[b])

        def wait_idx(cc, b):
            base = ebase + cc * CHUNK
            pltpu.make_async_copy(
                esrc_hbm.at[pl.ds(base, CHUNK)], SIDX[b], SEMI[b]).wait()
            pltpu.make_async_copy(
                edst_hbm.at[pl.ds(base, CHUNK)], DIDX[b], SEMI[b]).wait()

        def issue_gathers(b):
            pltpu.async_copy(wh_hbm.at[DIDX[b]], ROWS[b], SEMG[b])
            pltpu.async_copy(s1_hbm.at[SIDX[b]], SV1[b], SEMG[b])
            pltpu.async_copy(s2_hbm.at[DIDX[b]], SV2[b], SEMG[b])

        def wait_gathers(b):
            pltpu.make_async_copy(wh_hbm.at[DIDX[b]], ROWS[b],
                                  SEMG[b]).wait()
            pltpu.make_async_copy(s1_hbm.at[SIDX[b]], SV1[b],
                                  SEMG[b]).wait()
            pltpu.make_async_copy(s2_hbm.at[DIDX[b]], SV2[b],
                                  SEMG[b]).wait()

        def drain_scatter(b):
            pltpu.make_async_copy(ROWS[b], acc.at[SV1[b]], SEMS[b]).wait()

        def weigh_scale(b, psum):
            sv1, sv2, rows = SV1[b], SV2[b], ROWS[b]

            def group(g, acc_p):
                s1v = plsc.bitcast(sv1[pl.ds(g * 16, 16)], jnp.float32)
                e = s1v + sv2[pl.ds(g * 16, 16)]
                e = jnp.where(e >= 0, e, ALPHA * e)
                p16 = jnp.exp(e - c_shift)
                for j in range(16):
                    pe = p16[j]
                    ei = g * 16 + j
                    for k in range(GROUPS):
                        sl = pl.ds(k * 16, 16)
                        rows[ei, sl] = rows[ei, sl] * pe
                return acc_p + p16

            return lax.fori_loop(0, GROUPS, group, psum)

        def copy_idx(b):
            for g in range(GROUPS):
                SV1[b][pl.ds(g * 16, 16)] = SIDX[b][pl.ds(g * 16, 16)]

        def phase(cc, b, psum, drain=True, lead=True, load=True):
            bn = (b + 1) % 3
            if lead:
                wait_idx(cc + 1, bn)
            if drain:
                drain_scatter(bn)
            if lead:
                issue_gathers(bn)
            wait_gathers(b)
            psum = weigh_scale(b, psum)
            copy_idx(b)
            pltpu.async_copy(ROWS[b], acc.at[SV1[b]], SEMS[b], add=True)
            if load:
                load_idx(cc + 3, b)
            return psum

        load_idx(0, 0)
        load_idx(1, 1)
        load_idx(2, 2)
        wait_idx(0, 0)
        issue_gathers(0)

        psum = jnp.zeros((16,), jnp.float32)
        psum = phase(0, 0, psum, drain=False)
        psum = phase(1, 1, psum, drain=False)
        psum = phase(2, 2, psum)

        def triple(j, ps):
            c0 = 3 * j
            ps = phase(c0, 0, ps)
            ps = phase(c0 + 1, 1, ps)
            ps = phase(c0 + 2, 2, ps)
            return ps

        psum = lax.fori_loop(1, NFULL // 3 - 1, triple, psum)

        psum = phase(NFULL - 3, 0, psum, load=False)
        psum = phase(NFULL - 2, 1, psum, load=False)
        psum = phase(NFULL - 1, 2, psum, lead=False, load=False)

        tbase = ebase + NFULL * CHUNK
        pltpu.sync_copy(esrc_hbm.at[pl.ds(tbase, TAIL)],
                        sidx_a.at[pl.ds(0, TAIL)])
        pltpu.sync_copy(edst_hbm.at[pl.ds(tbase, TAIL)],
                        didx_a.at[pl.ds(0, TAIL)])
        issue_gathers(0)
        wait_gathers(0)
        e = (plsc.bitcast(sv1_a[pl.ds(0, 16)], jnp.float32)
             + sv2_a[pl.ds(0, 16)])
        e = jnp.where(e >= 0, e, ALPHA * e)
        tp = jnp.exp(e - c_shift)
        psum = psum + tp
        for j in range(16):
            pe = tp[j]
            for k in range(GROUPS):
                sl = pl.ds(k * 16, 16)
                rows_a[j, sl] = rows_a[j, sl] * pe

        def zero_rest(i, carry):
            for k in range(GROUPS):
                rows_a[i, pl.ds(k * 16, 16)] = zero16
            return carry

        lax.fori_loop(TAIL, CHUNK, zero_rest, 0)
        copy_idx(0)
        pltpu.sync_copy(rows_a, acc.at[sv1_a], add=True)
        drain_scatter(1)
        drain_scatter(2)

        plsc.subcore_barrier()

        pltpu.sync_copy(acc.at[pl.ds(sid * RSTRIPE, RSTRIPE)],
                        parts_hbm.at[cid, pl.ds(sid * RSTRIPE, RSTRIPE)])

        @pl.when(sid == NS - 1)
        def _out_tail():
            pltpu.sync_copy(acc.at[pl.ds(NS * RSTRIPE, RTAIL)],
                            parts_hbm.at[cid, pl.ds(NS * RSTRIPE, RTAIL)])

        rows_a[0, pl.ds(0, 16)] = psum
        pltpu.sync_copy(rows_a.at[0, pl.ds(0, 16)], psums_hbm.at[wid, 0])

    return sc_kernel


_sc_cache = None


def _get_sc_kernel():
    global _sc_cache
    if _sc_cache is None:
        _sc_cache = _make_sc_kernel()
    return _sc_cache


def _ep_body(parts_ref, psums_ref, o_ref):
    s = jnp.sum(psums_ref[...])
    v = (parts_ref[0] + parts_ref[1]) * (1.0 / s)
    o_ref[...] = jnp.where(v > 0, v, jnp.exp(v) - 1.0)


def _epilogue(parts, psums):
    return pl.pallas_call(
        _ep_body,
        grid=(N_NODES // BLK,),
        in_specs=[
            pl.BlockSpec((NC, BLK, OUT_F), lambda i: (0, i, 0)),
            pl.BlockSpec((NW, 1, 16), lambda i: (0, 0, 0)),
        ],
        out_specs=pl.BlockSpec((BLK, OUT_F), lambda i: (i, 0)),
        out_shape=jax.ShapeDtypeStruct((N_NODES, OUT_F), jnp.float32),
    )(parts, psums)


def kernel(x, edge_index, W, a):
    x = x.astype(jnp.float32)
    ei = edge_index.astype(jnp.int32)
    a_col = a[:, 0].astype(jnp.float32)
    At = jnp.zeros((IN_F, IN_F), jnp.float32)
    At = At.at[0, :].set(a_col[:IN_F]).at[1, :].set(a_col[IN_F:])
    Wh, s_all, c_arr = _matmul_scores(x, W.astype(jnp.float32), At)
    s1_bits = lax.bitcast_convert_type(s_all[0], jnp.int32)
    parts, psums = _get_sc_kernel()(Wh, s1_bits, s_all[1],
                                    ei[0], ei[1], c_arr)
    return _epilogue(parts, psums)
